# baseline (speedup 1.0000x reference)
import functools

import jax
import jax.numpy as jnp
from jax import lax
from jax.experimental import pallas as pl
from jax.experimental.pallas import tpu as pltpu
from jax.experimental.pallas import tpu_sc as plsc

B = 1024
KP1 = 512
D = 128
T = 0.07
MOM = 0.5

NC, NS, L = 2, 16, 16
NW = NC * NS
BPW = B // NW
KC = 128
NKC = KP1 // KC
NG = KC // L
DV = D // L

_mesh = plsc.VectorSubcoreMesh(
    core_axis_name="c", subcore_axis_name="s", num_cores=NC, num_subcores=NS)


def _worker_id():
    return lax.axis_index("s") * NC + lax.axis_index("c")


def _rsqrt16(v):
    i = plsc.bitcast(v, jnp.int32)
    i = jnp.int32(0x5F3759DF) - lax.shift_right_logical(i, 1)
    g = plsc.bitcast(i, jnp.float32)
    for _ in range(4):
        g = g * (1.5 - 0.5 * v * g * g)
    return g


@functools.partial(
    pl.kernel,
    out_type=(
        jax.ShapeDtypeStruct((B, KP1), jnp.float32),
        jax.ShapeDtypeStruct((B, KP1), jnp.float32),
    ),
    mesh=_mesh,
    compiler_params=pltpu.CompilerParams(needs_layout_passes=False),
    scratch_types=[
        pltpu.VMEM((BPW, D), jnp.float32),
        pltpu.VMEM((BPW, D), jnp.float32),
        pltpu.VMEM((BPW, KP1), jnp.int32),
        pltpu.VMEM((2, KC, D), jnp.float32),
        pltpu.VMEM((2, KC, D), jnp.float32),
        pltpu.VMEM((2, KP1), jnp.float32),
        pltpu.VMEM((2, KP1), jnp.float32),
        pltpu.VMEM((B,), jnp.int32),
        pltpu.VMEM((1, BPW), jnp.int32),
        pltpu.VMEM((BPW,), jnp.int32),
        pltpu.VMEM((BPW, D), jnp.float32),
        pltpu.VMEM((BPW, D), jnp.float32),
        pltpu.VMEM((BPW, D), jnp.float32),
        pltpu.VMEM((BPW, D), jnp.float32),
        pltpu.VMEM((BPW, D), jnp.float32),
        pltpu.VMEM((BPW, D), jnp.float32),
        pltpu.SemaphoreType.DMA((2,)),
        pltpu.SemaphoreType.DMA((2,)),
        pltpu.SemaphoreType.DMA((2,)),
        pltpu.SemaphoreType.DMA((2,)),
        pltpu.SemaphoreType.DMA,
        pltpu.SemaphoreType.DMA,
    ],
)
def _fused(l_hbm, ab_hbm, y_hbm, idx_hbm, ml_hbm, mab_hbm, ml_ref, mab_ref,
           outl_hbm, outab_hbm,
           lq_v, abq_v, idx_v, wl_v, wab_v, outl_rows, outab_rows,
           y_v, yc_v, bstar_v, m0l_v, m0ab_v, lst_v, abst_v, updl_v, updab_v,
           sem_l, sem_ab, sem_ol, sem_oab, sem_u1, sem_u2):
    wid = _worker_id()
    base = wid * BPW
    pltpu.sync_copy(l_hbm.at[pl.ds(base, BPW)], lq_v)
    pltpu.sync_copy(ab_hbm.at[pl.ds(base, BPW)], abq_v)
    pltpu.sync_copy(idx_hbm.at[pl.ds(base, BPW)], idx_v)
    pltpu.sync_copy(y_hbm.at[pl.ds(base, BPW)], yc_v.at[0])

    cm1 = pltpu.async_copy(ml_hbm.at[yc_v.at[0]], m0l_v, sem_u1)
    cm2 = pltpu.async_copy(mab_hbm.at[yc_v.at[0]], m0ab_v, sem_u2)

    iota = lax.iota(jnp.int32, L)
    row_idx = [g * L + iota for g in range(NG)]
    inv_t = jnp.float32(1.0 / T)

    NSUB = 4
    SUB = KC // NSUB

    def gather_refs(j, kc, buf):
        out = []
        for (mem, w, sem) in ((ml_hbm, wl_v, sem_l), (mab_hbm, wab_v, sem_ab)):
            for q in range(NSUB):
                isl = idx_v.at[j, pl.ds(kc * KC + q * SUB, SUB)]
                out.append((mem.at[isl],
                            w.at[buf, pl.ds(q * SUB, SUB), :],
                            sem.at[buf]))
        return out

    def issue(j, kc, buf):
        for (s, d, sem) in gather_refs(j, kc, buf):
            pltpu.async_copy(s, d, sem)

    def wait_gather(j, kc, buf, which):
        refs = gather_refs(j, kc, buf)
        half = which * NSUB
        for (s, d, sem) in refs[half:half + NSUB]:
            pltpu.make_async_copy(s, d, sem).wait()

    offs9 = 9 * iota

    def chunk_dot2(wlb, wabb, j, kc, s):
        qlj = lq_v.at[j]
        qabj = abq_v.at[j]

        def dbody(_, carry):
            dd = carry[0]
            accs = carry[1:]
            rot = (dd + offs9) & (D - 1)
            qab = plsc.load_gather(qabj, [rot])
            ql = plsc.load_gather(qlj, [rot])
            new = tuple(
                accs[g] + plsc.load_gather(wlb, [row_idx[g], rot]) * qab
                for g in range(NG)) + tuple(
                accs[NG + g] + plsc.load_gather(wabb, [row_idx[g], rot]) * ql
                for g in range(NG))
            return (dd + 1,) + new

        init = (jnp.zeros((L,), jnp.int32),) + tuple(
            jnp.zeros((L,), jnp.float32) for _ in range(2 * NG))
        fin = lax.fori_loop(0, D, dbody, init, unroll=2)
        for g in range(NG):
            outab_rows[s, pl.ds(kc * KC + g * L, L)] = fin[1 + g] * inv_t
            outl_rows[s, pl.ds(kc * KC + g * L, L)] = fin[1 + NG + g] * inv_t

    issue(0, 0, 0)

    def do_bpair(i, carry):
        for s in range(2):
            j = 2 * i + s
            b = base + j

            @pl.when(i >= 1)
            def _(s=s, i=i):
                pltpu.make_async_copy(
                    outl_rows.at[s], outl_hbm.at[base + 2 * (i - 1) + s],
                    sem_ol.at[s]).wait()
                pltpu.make_async_copy(
                    outab_rows.at[s], outab_hbm.at[base + 2 * (i - 1) + s],
                    sem_oab.at[s]).wait()

            for kc in range(NKC):
                buf = kc % 2
                if kc < NKC - 1:
                    issue(j, kc + 1, (kc + 1) % 2)
                else:
                    @pl.when(j + 1 < BPW)
                    def _(j=j):
                        issue(j + 1, 0, 0)
                wait_gather(j, kc, buf, 0)
                wait_gather(j, kc, buf, 1)
                chunk_dot2(wl_v.at[buf], wab_v.at[buf], j, kc, s)
            pltpu.async_copy(outl_rows.at[s], outl_hbm.at[b], sem_ol.at[s])
            pltpu.async_copy(outab_rows.at[s], outab_hbm.at[b], sem_oab.at[s])
        return carry

    lax.fori_loop(0, BPW // 2, do_bpair, 0)

    pltpu.sync_copy(y_hbm, y_v)
    neg1 = jnp.full((L,), -1, jnp.int32)

    def jbody(j, carry):
        b = base + j
        yb = plsc.load_gather(y_v, [jnp.full((L,), b, jnp.int32)])
        bv = jnp.full((L,), b, jnp.int32)

        def tbody(t, best):
            yt = y_v[pl.ds(t * L, L)]
            gidx = t * L + iota
            cand = jnp.where((yt == yb) & (gidx > bv), gidx, neg1)
            return lax.max(best, jnp.max(cand))

        bstar = lax.fori_loop(lax.div(b, L), B // L, tbody, b)
        plsc.store_scatter(
            bstar_v, [jnp.full((L,), j, jnp.int32)],
            jnp.full((L,), bstar, jnp.int32), mask=iota == 0)
        return carry

    lax.fori_loop(0, BPW, jbody, 0)

    cm1.wait()
    cm2.wait()
    c3 = pltpu.async_copy(l_hbm.at[bstar_v], lst_v, sem_u1)
    c4 = pltpu.async_copy(ab_hbm.at[bstar_v], abst_v, sem_u2)
    c3.wait()
    c4.wait()

    def ubody(j, carry):
        for (m0, st, upd) in ((m0l_v, lst_v, updl_v), (m0ab_v, abst_v, updab_v)):
            acc = jnp.zeros((L,), jnp.float32)
            pos = []
            for v in range(DV):
                p = m0[j, pl.ds(v * L, L)] * MOM + st[j, pl.ds(v * L, L)] * (1.0 - MOM)
                pos.append(p)
                acc = acc + p * p
            g = _rsqrt16(jnp.full((L,), jnp.sum(acc), jnp.float32))
            for v in range(DV):
                upd[j, pl.ds(v * L, L)] = pos[v] * g
        return carry

    lax.fori_loop(0, BPW, ubody, 0)

    c5 = pltpu.async_copy(updl_v, ml_ref.at[yc_v.at[0]], sem_u1)
    c6 = pltpu.async_copy(updab_v, mab_ref.at[yc_v.at[0]], sem_u2)
    c5.wait()
    c6.wait()

    for jj in (BPW - 2, BPW - 1):
        s = jj % 2
        pltpu.make_async_copy(
            outl_rows.at[s], outl_hbm.at[base + jj], sem_ol.at[s]).wait()
        pltpu.make_async_copy(
            outab_rows.at[s], outab_hbm.at[base + jj], sem_oab.at[s]).wait()


def kernel(l, ab, y, idx, memory_l, memory_ab):
    y32 = y.astype(jnp.int32)
    idx32 = idx.astype(jnp.int32)
    ml_ref = jax.new_ref(memory_l)
    mab_ref = jax.new_ref(memory_ab)
    out_l, out_ab = _fused(l, ab, y32, idx32, memory_l, memory_ab,
                           ml_ref, mab_ref)
    return (out_l[..., None], out_ab[..., None],
            jax.freeze(ml_ref), jax.freeze(mab_ref))


# speedup vs baseline: 1.0560x; 1.0560x over previous
import functools

import jax
import jax.numpy as jnp
from jax import lax
from jax.experimental import pallas as pl
from jax.experimental.pallas import tpu as pltpu
from jax.experimental.pallas import tpu_sc as plsc

B = 1024
KP1 = 512
D = 128
T = 0.07
MOM = 0.5

NC, NS, L = 2, 16, 16
NW = NC * NS
BPW = B // NW
KC = 128
NKC = KP1 // KC
NG = KC // L
DV = D // L

_mesh = plsc.VectorSubcoreMesh(
    core_axis_name="c", subcore_axis_name="s", num_cores=NC, num_subcores=NS)


def _worker_id():
    return lax.axis_index("s") * NC + lax.axis_index("c")


def _rsqrt16(v):
    i = plsc.bitcast(v, jnp.int32)
    i = jnp.int32(0x5F3759DF) - lax.shift_right_logical(i, 1)
    g = plsc.bitcast(i, jnp.float32)
    for _ in range(4):
        g = g * (1.5 - 0.5 * v * g * g)
    return g


@functools.partial(
    pl.kernel,
    out_type=(
        jax.ShapeDtypeStruct((B, KP1), jnp.float32),
        jax.ShapeDtypeStruct((B, KP1), jnp.float32),
    ),
    mesh=_mesh,
    compiler_params=pltpu.CompilerParams(needs_layout_passes=False),
    scratch_types=[
        pltpu.VMEM((BPW, D), jnp.float32),
        pltpu.VMEM((BPW, D), jnp.float32),
        pltpu.VMEM((BPW, KP1), jnp.int32),
        pltpu.VMEM((2, KC, D), jnp.float32),
        pltpu.VMEM((2, KC, D), jnp.float32),
        pltpu.VMEM((2, KP1), jnp.float32),
        pltpu.VMEM((2, KP1), jnp.float32),
        pltpu.SemaphoreType.DMA((2,)),
        pltpu.SemaphoreType.DMA((2,)),
        pltpu.SemaphoreType.DMA((2,)),
        pltpu.SemaphoreType.DMA((2,)),
    ],
)
def _gather_dot(l_hbm, ab_hbm, idx_hbm, ml_hbm, mab_hbm,
                outl_hbm, outab_hbm,
                lq_v, abq_v, idx_v, wl_v, wab_v, outl_rows, outab_rows,
                sem_l, sem_ab, sem_ol, sem_oab):
    wid = _worker_id()
    base = wid * BPW
    pltpu.sync_copy(l_hbm.at[pl.ds(base, BPW)], lq_v)
    pltpu.sync_copy(ab_hbm.at[pl.ds(base, BPW)], abq_v)
    pltpu.sync_copy(idx_hbm.at[pl.ds(base, BPW)], idx_v)

    iota = lax.iota(jnp.int32, L)
    row_idx = [g * L + iota for g in range(NG)]
    inv_t = jnp.float32(1.0 / T)

    NSUB = 4
    SUB = KC // NSUB

    def gather_refs(j, kc, buf):
        out = []
        for (mem, w, sem) in ((ml_hbm, wl_v, sem_l), (mab_hbm, wab_v, sem_ab)):
            for q in range(NSUB):
                isl = idx_v.at[j, pl.ds(kc * KC + q * SUB, SUB)]
                out.append((mem.at[isl],
                            w.at[buf, pl.ds(q * SUB, SUB), :],
                            sem.at[buf]))
        return out

    def issue(j, kc, buf):
        for (s, d, sem) in gather_refs(j, kc, buf):
            pltpu.async_copy(s, d, sem)

    def wait_gather(j, kc, buf, which):
        refs = gather_refs(j, kc, buf)
        half = which * NSUB
        for (s, d, sem) in refs[half:half + NSUB]:
            pltpu.make_async_copy(s, d, sem).wait()

    offs9 = 9 * iota

    def chunk_dot2(wlb, wabb, j, kc, s):
        qlj = lq_v.at[j]
        qabj = abq_v.at[j]

        def dbody(_, carry):
            dd = carry[0]
            accs = carry[1:]
            rot = (dd + offs9) & (D - 1)
            qab = plsc.load_gather(qabj, [rot])
            ql = plsc.load_gather(qlj, [rot])
            new = tuple(
                accs[g] + plsc.load_gather(wlb, [row_idx[g], rot]) * qab
                for g in range(NG)) + tuple(
                accs[NG + g] + plsc.load_gather(wabb, [row_idx[g], rot]) * ql
                for g in range(NG))
            return (dd + 1,) + new

        init = (jnp.zeros((L,), jnp.int32),) + tuple(
            jnp.zeros((L,), jnp.float32) for _ in range(2 * NG))
        fin = lax.fori_loop(0, D, dbody, init, unroll=2)
        for g in range(NG):
            outab_rows[s, pl.ds(kc * KC + g * L, L)] = fin[1 + g] * inv_t
            outl_rows[s, pl.ds(kc * KC + g * L, L)] = fin[1 + NG + g] * inv_t

    issue(0, 0, 0)

    def do_bpair(i, carry):
        for s in range(2):
            j = 2 * i + s
            b = base + j

            @pl.when(i >= 1)
            def _(s=s, i=i):
                pltpu.make_async_copy(
                    outl_rows.at[s], outl_hbm.at[base + 2 * (i - 1) + s],
                    sem_ol.at[s]).wait()
                pltpu.make_async_copy(
                    outab_rows.at[s], outab_hbm.at[base + 2 * (i - 1) + s],
                    sem_oab.at[s]).wait()

            for kc in range(NKC):
                buf = kc % 2
                if kc < NKC - 1:
                    issue(j, kc + 1, (kc + 1) % 2)
                else:
                    @pl.when(j + 1 < BPW)
                    def _(j=j):
                        issue(j + 1, 0, 0)
                wait_gather(j, kc, buf, 0)
                wait_gather(j, kc, buf, 1)
                chunk_dot2(wl_v.at[buf], wab_v.at[buf], j, kc, s)
            pltpu.async_copy(outl_rows.at[s], outl_hbm.at[b], sem_ol.at[s])
            pltpu.async_copy(outab_rows.at[s], outab_hbm.at[b], sem_oab.at[s])
        return carry

    lax.fori_loop(0, BPW // 2, do_bpair, 0)
    for jj in (BPW - 2, BPW - 1):
        s = jj % 2
        pltpu.make_async_copy(
            outl_rows.at[s], outl_hbm.at[base + jj], sem_ol.at[s]).wait()
        pltpu.make_async_copy(
            outab_rows.at[s], outab_hbm.at[base + jj], sem_oab.at[s]).wait()


@functools.partial(
    pl.kernel,
    out_type=(),
    mesh=_mesh,
    compiler_params=pltpu.CompilerParams(needs_layout_passes=False),
    scratch_types=[
        pltpu.VMEM((B,), jnp.int32),
        pltpu.VMEM((1, BPW), jnp.int32),
        pltpu.VMEM((BPW,), jnp.int32),
        pltpu.VMEM((BPW, D), jnp.float32),
        pltpu.VMEM((BPW, D), jnp.float32),
        pltpu.VMEM((BPW, D), jnp.float32),
        pltpu.VMEM((BPW, D), jnp.float32),
        pltpu.VMEM((BPW, D), jnp.float32),
        pltpu.VMEM((BPW, D), jnp.float32),
        pltpu.SemaphoreType.DMA,
        pltpu.SemaphoreType.DMA,
    ],
)
def _update(l_hbm, ab_hbm, y_hbm, ml_in, mab_in, ml_ref, mab_ref,
            y_v, yc_v, bstar_v, m0l_v, m0ab_v, lst_v, abst_v,
            updl_v, updab_v, sem1, sem2):
    wid = _worker_id()
    base = wid * BPW
    pltpu.sync_copy(y_hbm, y_v)
    pltpu.sync_copy(y_hbm.at[pl.ds(base, BPW)], yc_v.at[0])

    c1 = pltpu.async_copy(ml_in.at[yc_v.at[0]], m0l_v, sem1)
    c2 = pltpu.async_copy(mab_in.at[yc_v.at[0]], m0ab_v, sem2)

    iota = lax.iota(jnp.int32, L)
    neg1 = jnp.full((L,), -1, jnp.int32)

    def jbody(j, carry):
        b = base + j
        yb = plsc.load_gather(y_v, [jnp.full((L,), b, jnp.int32)])
        bv = jnp.full((L,), b, jnp.int32)

        def tbody(t, best):
            yt = y_v[pl.ds(t * L, L)]
            gidx = t * L + iota
            cand = jnp.where((yt == yb) & (gidx > bv), gidx, neg1)
            return lax.max(best, jnp.max(cand))

        bstar = lax.fori_loop(lax.div(b, L), B // L, tbody, b)
        plsc.store_scatter(
            bstar_v, [jnp.full((L,), j, jnp.int32)],
            jnp.full((L,), bstar, jnp.int32), mask=iota == 0)
        return carry

    lax.fori_loop(0, BPW, jbody, 0)

    c1.wait()
    c2.wait()
    c3 = pltpu.async_copy(l_hbm.at[bstar_v], lst_v, sem1)
    c4 = pltpu.async_copy(ab_hbm.at[bstar_v], abst_v, sem2)
    c3.wait()
    c4.wait()

    def ubody(j, carry):
        for (m0, st, upd) in ((m0l_v, lst_v, updl_v), (m0ab_v, abst_v, updab_v)):
            acc = jnp.zeros((L,), jnp.float32)
            pos = []
            for v in range(DV):
                p = m0[j, pl.ds(v * L, L)] * MOM + st[j, pl.ds(v * L, L)] * (1.0 - MOM)
                pos.append(p)
                acc = acc + p * p
            g = _rsqrt16(jnp.full((L,), jnp.sum(acc), jnp.float32))
            for v in range(DV):
                upd[j, pl.ds(v * L, L)] = pos[v] * g
        return carry

    lax.fori_loop(0, BPW, ubody, 0)

    c5 = pltpu.async_copy(updl_v, ml_ref.at[yc_v.at[0]], sem1)
    c6 = pltpu.async_copy(updab_v, mab_ref.at[yc_v.at[0]], sem2)
    c5.wait()
    c6.wait()


def kernel(l, ab, y, idx, memory_l, memory_ab):
    y32 = y.astype(jnp.int32)
    idx32 = idx.astype(jnp.int32)
    out_l, out_ab = _gather_dot(l, ab, idx32, memory_l, memory_ab)
    ml_ref = jax.new_ref(memory_l)
    mab_ref = jax.new_ref(memory_ab)
    _update(l, ab, y32, memory_l, memory_ab, ml_ref, mab_ref)
    return (out_l[..., None], out_ab[..., None],
            jax.freeze(ml_ref), jax.freeze(mab_ref))
